# baseline (speedup 1.0000x reference)
import functools

import jax
import jax.numpy as jnp
from jax import lax
from jax.experimental import pallas as pl
from jax.experimental.pallas import tpu as pltpu
from jax.experimental.pallas import tpu_sc as plsc

_TEMP = 0.1
_MOM = 0.1
_V = 100000
_D = 64
_B = 1024
_BR = 2000
_NB = _V // _BR
_C2 = float(1.4426950408889634 / _TEMP)


_G_NW = 32
_G_PER = _B // _G_NW


def _gather_body(lab_ref, feat_ref, g_ref, idx_v, rows_v, sem):
    c = lax.axis_index("c")
    s = lax.axis_index("s")
    wid = s * 2 + c
    base = wid * _G_PER
    pltpu.sync_copy(lab_ref.at[pl.ds(base, _G_PER)], idx_v)
    pltpu.async_copy(feat_ref.at[idx_v], rows_v, sem).wait()
    pltpu.sync_copy(rows_v, g_ref.at[pl.ds(base, _G_PER)])


@functools.cache
def _sc_gather():
    return pl.kernel(
        _gather_body,
        out_type=jax.ShapeDtypeStruct((_B, _D), jnp.float32),
        mesh=plsc.VectorSubcoreMesh(core_axis_name="c", subcore_axis_name="s"),
        scratch_types=[
            pltpu.VMEM((_G_PER,), jnp.int32),
            pltpu.VMEM((_G_PER, _D), jnp.float32),
            pltpu.SemaphoreType.DMA,
        ],
        compiler_params=pltpu.CompilerParams(use_tc_tiling_on_sc=False),
    )


def _main_body(xT_ref, gT_ref, lc_ref, lr_ref, f_ref, p_ref, updT_ref,
               xeT_ref, s_ref, down_ref):
    i = pl.program_id(0)

    @pl.when(i == 0)
    def _init():
        xT = xT_ref[...]
        n = jnp.sqrt(jnp.sum(xT * xT, axis=0, keepdims=True))
        xnT = xT / (n + 1e-12)
        xeT_ref[...] = xnT * _C2
        down_ref[...] = jnp.zeros_like(down_ref)
        gT = gT_ref[...]
        u = _MOM * gT + (1.0 - _MOM) * xnT
        un = jnp.sqrt(jnp.sum(u * u, axis=0, keepdims=True))
        u = u / (un + 1e-12)
        lc = lc_ref[...]
        lr = lr_ref[0:1, :]
        eq = lc == lr
        ii = lax.broadcasted_iota(jnp.int32, (_B, _B), 0)
        lo = jnp.max(jnp.where(eq, ii, -1), axis=0, keepdims=True)
        q = (ii == lo).astype(jnp.float32)
        updT_ref[...] = jnp.dot(u, q, precision=lax.Precision.HIGHEST)

    sm = jnp.where(i > 0, s_ref[...], -1e30)
    e = jnp.exp2(sm)
    ones = jnp.ones((1, _BR), jnp.float32)
    down_ref[...] += jnp.dot(ones, e, precision=lax.Precision.HIGHEST)
    s_ref[...] = jnp.dot(f_ref[...], xeT_ref[...])

    @pl.when(i == _NB)
    def _fin():
        dots = jnp.sum(xeT_ref[...] * gT_ref[...], axis=0, keepdims=True)
        p_ref[...] = jnp.exp2(dots) / down_ref[...]


_main_call = pl.pallas_call(
    _main_body,
    grid=(_NB + 1,),
    in_specs=[
        pl.BlockSpec((_D, _B), lambda i: (0, 0)),
        pl.BlockSpec((_D, _B), lambda i: (0, 0)),
        pl.BlockSpec((_B, 1), lambda i: (0, 0)),
        pl.BlockSpec((8, _B), lambda i: (0, 0)),
        pl.BlockSpec((_BR, _D), lambda i: (jnp.minimum(i, _NB - 1), 0)),
    ],
    out_specs=[
        pl.BlockSpec((1, _B), lambda i: (0, 0)),
        pl.BlockSpec((_D, _B), lambda i: (0, 0)),
    ],
    out_shape=[
        jax.ShapeDtypeStruct((1, _B), jnp.float32),
        jax.ShapeDtypeStruct((_D, _B), jnp.float32),
    ],
    scratch_shapes=[
        pltpu.VMEM((_D, _B), jnp.float32),
        pltpu.VMEM((_BR, _B), jnp.float32),
        pltpu.VMEM((1, _B), jnp.float32),
    ],
    compiler_params=pltpu.CompilerParams(
        dimension_semantics=("arbitrary",),
    ),
)


_S_NW = 16
_S_ROWS = _V // _S_NW
_S_CHUNK = 1250
_S_NCH = _S_ROWS // _S_CHUNK
_S_PER = _B // _S_NW


def _scatter_body(feat_ref, upd_ref, lab_ref, out_ref, buf, idx_v, rows_v,
                  sem):
    s = lax.axis_index("s")
    row0 = s * _S_ROWS
    for c in range(_S_NCH):
        b = row0 + c * _S_CHUNK
        pltpu.sync_copy(feat_ref.at[pl.ds(b, _S_CHUNK)], buf)
        pltpu.sync_copy(buf, out_ref.at[pl.ds(b, _S_CHUNK)])
    plsc.subcore_barrier()
    i0 = s * _S_PER
    pltpu.sync_copy(lab_ref.at[pl.ds(i0, _S_PER)], idx_v)
    pltpu.sync_copy(upd_ref.at[pl.ds(i0, _S_PER)], rows_v)
    pltpu.async_copy(rows_v, out_ref.at[idx_v], sem).wait()


@functools.cache
def _sc_scatter():
    return pl.kernel(
        _scatter_body,
        out_type=jax.ShapeDtypeStruct((_V, _D), jnp.float32),
        mesh=plsc.VectorSubcoreMesh(
            core_axis_name="c", subcore_axis_name="s", num_cores=1
        ),
        scratch_types=[
            pltpu.VMEM((_S_CHUNK, _D), jnp.float32),
            pltpu.VMEM((_S_PER,), jnp.int32),
            pltpu.VMEM((_S_PER, _D), jnp.float32),
            pltpu.SemaphoreType.DMA,
        ],
        compiler_params=pltpu.CompilerParams(use_tc_tiling_on_sc=False),
    )


def kernel(inputs, labels, features):
    labels = labels.astype(jnp.int32)
    g = _sc_gather()(labels, features)
    p2, updT = _main_call(
        inputs.T,
        g.T,
        labels[:, None],
        jnp.broadcast_to(labels[None, :], (8, _B)),
        features,
    )
    new_mem = _sc_scatter()(features, updT.T, labels)
    return p2.reshape(_B), new_mem


# speedup vs baseline: 1.6743x; 1.6743x over previous
import functools

import jax
import jax.numpy as jnp
from jax import lax
from jax.experimental import pallas as pl
from jax.experimental.pallas import tpu as pltpu
from jax.experimental.pallas import tpu_sc as plsc

_TEMP = 0.1
_MOM = 0.1
_V = 100000
_D = 64
_B = 1024
_BR = 2000
_NB = _V // _BR
_C2 = float(1.4426950408889634 / _TEMP)


_G_NW = 32
_G_PER = _B // _G_NW


def _gather_body(lab_ref, feat_ref, g_ref, idx_v, rows_v, sem):
    c = lax.axis_index("c")
    s = lax.axis_index("s")
    wid = s * 2 + c
    base = wid * _G_PER
    pltpu.sync_copy(lab_ref.at[pl.ds(base, _G_PER)], idx_v)
    pltpu.async_copy(feat_ref.at[idx_v], rows_v, sem).wait()
    pltpu.sync_copy(rows_v, g_ref.at[pl.ds(base, _G_PER)])


@functools.cache
def _sc_gather():
    return pl.kernel(
        _gather_body,
        out_type=jax.ShapeDtypeStruct((_B, _D), jnp.float32),
        mesh=plsc.VectorSubcoreMesh(core_axis_name="c", subcore_axis_name="s"),
        scratch_types=[
            pltpu.VMEM((_G_PER,), jnp.int32),
            pltpu.VMEM((_G_PER, _D), jnp.float32),
            pltpu.SemaphoreType.DMA,
        ],
        compiler_params=pltpu.CompilerParams(use_tc_tiling_on_sc=False),
    )


def _main_body(xT_ref, gT_ref, lc_ref, lr_ref, f_ref, p_ref, updT_ref,
               xeT_ref, s_ref, down_ref):
    i = pl.program_id(0)

    @pl.when(i == 0)
    def _init():
        xT = xT_ref[...]
        n = jnp.sqrt(jnp.sum(xT * xT, axis=0, keepdims=True))
        xnT = xT / (n + 1e-12)
        xeT_ref[...] = xnT * _C2
        down_ref[...] = jnp.zeros_like(down_ref)
        gT = gT_ref[...]
        u = _MOM * gT + (1.0 - _MOM) * xnT
        un = jnp.sqrt(jnp.sum(u * u, axis=0, keepdims=True))
        u = u / (un + 1e-12)
        lc = lc_ref[...]
        lr = lr_ref[0:1, :]
        eq = lc == lr
        ii = lax.broadcasted_iota(jnp.int32, (_B, _B), 0)
        lo = jnp.max(jnp.where(eq, ii, -1), axis=0, keepdims=True)
        q = (ii == lo).astype(jnp.float32)
        updT_ref[...] = jnp.dot(u, q, precision=lax.Precision.HIGHEST)

    sm = jnp.where(i > 0, s_ref[...], -1e30)
    e = jnp.exp2(sm)
    down_ref[...] += jnp.sum(e.reshape(_BR // 8, 8, _B), axis=0)
    s_ref[...] = jnp.dot(f_ref[...], xeT_ref[...])

    @pl.when(i == _NB)
    def _fin():
        dots = jnp.sum(xeT_ref[...] * gT_ref[...], axis=0, keepdims=True)
        down = jnp.sum(down_ref[...], axis=0, keepdims=True)
        p_ref[...] = jnp.exp2(dots) / down


_main_call = pl.pallas_call(
    _main_body,
    grid=(_NB + 1,),
    in_specs=[
        pl.BlockSpec((_D, _B), lambda i: (0, 0)),
        pl.BlockSpec((_D, _B), lambda i: (0, 0)),
        pl.BlockSpec((_B, 1), lambda i: (0, 0)),
        pl.BlockSpec((8, _B), lambda i: (0, 0)),
        pl.BlockSpec((_BR, _D), lambda i: (jnp.minimum(i, _NB - 1), 0)),
    ],
    out_specs=[
        pl.BlockSpec((1, _B), lambda i: (0, 0)),
        pl.BlockSpec((_D, _B), lambda i: (0, 0)),
    ],
    out_shape=[
        jax.ShapeDtypeStruct((1, _B), jnp.float32),
        jax.ShapeDtypeStruct((_D, _B), jnp.float32),
    ],
    scratch_shapes=[
        pltpu.VMEM((_D, _B), jnp.float32),
        pltpu.VMEM((_BR, _B), jnp.float32),
        pltpu.VMEM((8, _B), jnp.float32),
    ],
    compiler_params=pltpu.CompilerParams(
        dimension_semantics=("arbitrary",),
    ),
)


_S_NW = 16
_S_ROWS = _V // _S_NW
_S_CHUNK = 1250
_S_NCH = _S_ROWS // _S_CHUNK
_S_PER = _B // _S_NW


def _scatter_body(feat_ref, upd_ref, lab_ref, out_ref, buf, idx_v, rows_v,
                  sem):
    s = lax.axis_index("s")
    row0 = s * _S_ROWS
    for c in range(_S_NCH):
        b = row0 + c * _S_CHUNK
        pltpu.sync_copy(feat_ref.at[pl.ds(b, _S_CHUNK)], buf)
        pltpu.sync_copy(buf, out_ref.at[pl.ds(b, _S_CHUNK)])
    plsc.subcore_barrier()
    i0 = s * _S_PER
    pltpu.sync_copy(lab_ref.at[pl.ds(i0, _S_PER)], idx_v)
    pltpu.sync_copy(upd_ref.at[pl.ds(i0, _S_PER)], rows_v)
    pltpu.async_copy(rows_v, out_ref.at[idx_v], sem).wait()


@functools.cache
def _sc_scatter():
    return pl.kernel(
        _scatter_body,
        out_type=jax.ShapeDtypeStruct((_V, _D), jnp.float32),
        mesh=plsc.VectorSubcoreMesh(
            core_axis_name="c", subcore_axis_name="s", num_cores=1
        ),
        scratch_types=[
            pltpu.VMEM((_S_CHUNK, _D), jnp.float32),
            pltpu.VMEM((_S_PER,), jnp.int32),
            pltpu.VMEM((_S_PER, _D), jnp.float32),
            pltpu.SemaphoreType.DMA,
        ],
        compiler_params=pltpu.CompilerParams(use_tc_tiling_on_sc=False),
    )


def kernel(inputs, labels, features):
    labels = labels.astype(jnp.int32)
    g = _sc_gather()(labels, features)
    p2, updT = _main_call(
        inputs.T,
        g.T,
        labels[:, None],
        jnp.broadcast_to(labels[None, :], (8, _B)),
        features,
    )
    new_mem = _sc_scatter()(features, updT.T, labels)
    return p2.reshape(_B), new_mem


# speedup vs baseline: 1.7677x; 1.0558x over previous
import functools

import jax
import jax.numpy as jnp
from jax import lax
from jax.experimental import pallas as pl
from jax.experimental.pallas import tpu as pltpu
from jax.experimental.pallas import tpu_sc as plsc

_TEMP = 0.1
_MOM = 0.1
_V = 100000
_D = 64
_B = 1024
_BR = 2000
_NB = _V // _BR
_C2 = float(1.4426950408889634 / _TEMP)


_G_NW = 32
_G_PER = _B // _G_NW


def _gather_body(lab_ref, feat_ref, g8_ref, idx_v, rows_v, sem):
    c = lax.axis_index("c")
    s = lax.axis_index("s")
    wid = s * 2 + c
    base = wid * _G_PER
    pltpu.sync_copy(lab_ref.at[pl.ds(base, _G_PER)], idx_v)

    for grp in range(_G_PER // 16):
        lab_vec = idx_v[pl.ds(grp * 16, 16)]
        for j in range(16):
            k = grp * 16 + j
            tb = pl.multiple_of((lab_vec[j] >> 3) * 8, 8)
            pltpu.async_copy(feat_ref.at[pl.ds(tb, 8)],
                             rows_v.at[pl.ds(k * 8, 8)], sem)
    pltpu.make_async_copy(feat_ref.at[pl.ds(0, _G_PER * 8)], rows_v,
                          sem).wait()
    pltpu.sync_copy(rows_v, g8_ref.at[pl.ds(base * 8, _G_PER * 8)])


@functools.cache
def _sc_gather():
    return pl.kernel(
        _gather_body,
        out_type=jax.ShapeDtypeStruct((_B * 8, _D), jnp.float32),
        mesh=plsc.VectorSubcoreMesh(core_axis_name="c", subcore_axis_name="s"),
        scratch_types=[
            pltpu.VMEM((_G_PER,), jnp.int32),
            pltpu.VMEM((_G_PER * 8, _D), jnp.float32),
            pltpu.SemaphoreType.DMA,
        ],
    )


def _main_body(xT_ref, g8r_ref, lc_ref, lr_ref, f_ref, p_ref, t8_ref,
               xeT_ref, gT_ref, s_ref, down_ref):
    i = pl.program_id(0)

    @pl.when(i == 0)
    def _init():
        xT = xT_ref[...]
        n = jnp.sqrt(jnp.sum(xT * xT, axis=0, keepdims=True))
        xnT = xT / (n + 1e-12)
        xeT_ref[...] = xnT * _C2
        down_ref[...] = jnp.zeros_like(down_ref)
        lc = lc_ref[...]
        lr = lr_ref[0:1, :]
        slot = lr - ((lr >> 3) << 3)
        gT = jnp.zeros((_D, _B), jnp.float32)
        for r in range(8):
            gT = gT + jnp.where(slot == r, g8r_ref[r], 0.0)
        gT_ref[...] = gT
        u = _MOM * gT + (1.0 - _MOM) * xnT
        un = jnp.sqrt(jnp.sum(u * u, axis=0, keepdims=True))
        u = u / (un + 1e-12)
        gb = (lr >> 3) << 3
        ii = lax.broadcasted_iota(jnp.int32, (_B, _B), 0)
        for r in range(8):
            eq_r = lc == (gb + r)
            lo_r = jnp.max(jnp.where(eq_r, ii, -1), axis=0, keepdims=True)
            q_r = (ii == lo_r).astype(jnp.float32)
            val_r = jnp.dot(u, q_r)
            t8_ref[r * _D:(r + 1) * _D, :] = jnp.where(
                lo_r >= 0, val_r, g8r_ref[r])

    sm = jnp.where(i > 0, s_ref[...], -1e30)
    e = jnp.exp2(sm)
    down_ref[...] += jnp.sum(e.reshape(_BR // 8, 8, _B), axis=0)
    s_ref[...] = jnp.dot(f_ref[...], xeT_ref[...])

    @pl.when(i == _NB)
    def _fin():
        dots = jnp.sum(xeT_ref[...] * gT_ref[...], axis=0, keepdims=True)
        down = jnp.sum(down_ref[...], axis=0, keepdims=True)
        p_ref[...] = jnp.exp2(dots) / down


_main_call = pl.pallas_call(
    _main_body,
    grid=(_NB + 1,),
    in_specs=[
        pl.BlockSpec((_D, _B), lambda i: (0, 0)),
        pl.BlockSpec((8, _D, _B), lambda i: (0, 0, 0)),
        pl.BlockSpec((_B, 1), lambda i: (0, 0)),
        pl.BlockSpec((8, _B), lambda i: (0, 0)),
        pl.BlockSpec((_BR, _D), lambda i: (jnp.minimum(i, _NB - 1), 0)),
    ],
    out_specs=[
        pl.BlockSpec((1, _B), lambda i: (0, 0)),
        pl.BlockSpec((8 * _D, _B), lambda i: (0, 0)),
    ],
    out_shape=[
        jax.ShapeDtypeStruct((1, _B), jnp.float32),
        jax.ShapeDtypeStruct((8 * _D, _B), jnp.float32),
    ],
    scratch_shapes=[
        pltpu.VMEM((_D, _B), jnp.float32),
        pltpu.VMEM((_D, _B), jnp.float32),
        pltpu.VMEM((_BR, _B), jnp.float32),
        pltpu.VMEM((8, _B), jnp.float32),
    ],
    compiler_params=pltpu.CompilerParams(
        dimension_semantics=("arbitrary",),
    ),
)


_S_NW = 16
_S_ROWS = 6248
_S_CHUNK = 496
_S_NCH = 12
_S_TAIL = _S_ROWS - _S_NCH * _S_CHUNK
_S_REM_BASE = _S_NW * _S_ROWS
_S_REM = _V - _S_REM_BASE
_S_PER = _B // _S_NW


def _scatter_body(feat_ref, upd_ref, lab_ref, out_ref, buf, idx_v, rows_v,
                  sem):
    s = lax.axis_index("s")
    row0 = s * _S_ROWS
    for c in range(_S_NCH):
        b = row0 + c * _S_CHUNK
        pltpu.sync_copy(feat_ref.at[pl.ds(b, _S_CHUNK)], buf)
        pltpu.sync_copy(buf, out_ref.at[pl.ds(b, _S_CHUNK)])
    bt = row0 + _S_NCH * _S_CHUNK
    pltpu.sync_copy(feat_ref.at[pl.ds(bt, _S_TAIL)],
                    buf.at[pl.ds(0, _S_TAIL)])
    pltpu.sync_copy(buf.at[pl.ds(0, _S_TAIL)], out_ref.at[pl.ds(bt, _S_TAIL)])

    @pl.when(s == 0)
    def _rem():
        pltpu.sync_copy(feat_ref.at[pl.ds(_S_REM_BASE, _S_REM)],
                        buf.at[pl.ds(0, _S_REM)])
        pltpu.sync_copy(buf.at[pl.ds(0, _S_REM)],
                        out_ref.at[pl.ds(_S_REM_BASE, _S_REM)])

    plsc.subcore_barrier()
    i0 = s * _S_PER
    pltpu.sync_copy(lab_ref.at[pl.ds(i0, _S_PER)], idx_v)
    pltpu.sync_copy(upd_ref.at[pl.ds(i0 * 8, _S_PER * 8)], rows_v)

    for grp in range(_S_PER // 16):
        lab_vec = idx_v[pl.ds(grp * 16, 16)]
        for j in range(16):
            k = grp * 16 + j
            tb = pl.multiple_of((lab_vec[j] >> 3) * 8, 8)
            pltpu.async_copy(rows_v.at[pl.ds(k * 8, 8)],
                             out_ref.at[pl.ds(tb, 8)], sem)
    pltpu.make_async_copy(feat_ref.at[pl.ds(0, _S_PER * 8)], rows_v,
                          sem).wait()


@functools.cache
def _sc_scatter():
    return pl.kernel(
        _scatter_body,
        out_type=jax.ShapeDtypeStruct((_V, _D), jnp.float32),
        mesh=plsc.VectorSubcoreMesh(
            core_axis_name="c", subcore_axis_name="s", num_cores=1
        ),
        scratch_types=[
            pltpu.VMEM((_S_CHUNK, _D), jnp.float32),
            pltpu.VMEM((_S_PER,), jnp.int32),
            pltpu.VMEM((_S_PER * 8, _D), jnp.float32),
            pltpu.SemaphoreType.DMA,
        ],
    )


def kernel(inputs, labels, features):
    labels = labels.astype(jnp.int32)
    g8 = _sc_gather()(labels, features)
    g8r = jnp.transpose(g8.reshape(_B, 8, _D), (1, 2, 0))
    p2, t8 = _main_call(
        inputs.T,
        g8r,
        labels[:, None],
        jnp.broadcast_to(labels[None, :], (8, _B)),
        features,
    )
    upd8 = t8.T.reshape(_B * 8, _D)
    new_mem = _sc_scatter()(features, upd8, labels)
    return p2.reshape(_B), new_mem


# speedup vs baseline: 2.3056x; 1.3043x over previous
import functools

import jax
import jax.numpy as jnp
from jax import lax
from jax.experimental import pallas as pl
from jax.experimental.pallas import tpu as pltpu
from jax.experimental.pallas import tpu_sc as plsc

_TEMP = 0.1
_MOM = 0.1
_V = 100000
_D = 64
_B = 1024
_BR = 2000
_NB = _V // _BR
_C2 = float(1.4426950408889634 / _TEMP)


_G_NW = 32
_G_PER = _B // _G_NW


def _gather_body(lab_ref, feat_ref, g8_ref, idx_v, rows_v, sem):
    c = lax.axis_index("c")
    s = lax.axis_index("s")
    wid = s * 2 + c
    base = wid * _G_PER
    pltpu.sync_copy(lab_ref.at[pl.ds(base, _G_PER)], idx_v)

    for grp in range(_G_PER // 16):
        lab_vec = idx_v[pl.ds(grp * 16, 16)]
        for j in range(16):
            k = grp * 16 + j
            tb = pl.multiple_of((lab_vec[j] >> 3) * 8, 8)
            pltpu.async_copy(feat_ref.at[pl.ds(tb, 8)],
                             rows_v.at[pl.ds(k * 8, 8)], sem)
    pltpu.make_async_copy(feat_ref.at[pl.ds(0, _G_PER * 8)], rows_v,
                          sem).wait()
    pltpu.sync_copy(rows_v, g8_ref.at[pl.ds(base * 8, _G_PER * 8)])


@functools.cache
def _sc_gather():
    return pl.kernel(
        _gather_body,
        out_type=jax.ShapeDtypeStruct((_B * 8, _D), jnp.float32),
        mesh=plsc.VectorSubcoreMesh(core_axis_name="c", subcore_axis_name="s"),
        scratch_types=[
            pltpu.VMEM((_G_PER,), jnp.int32),
            pltpu.VMEM((_G_PER * 8, _D), jnp.float32),
            pltpu.SemaphoreType.DMA,
        ],
    )


def _main_body(xT_ref, g8r_ref, lc_ref, lr_ref, f_ref, p_ref, t8_ref,
               mem_ref, xeT_ref, gT_ref, s_ref, down_ref):
    i = pl.program_id(0)

    @pl.when(i == 0)
    def _init():
        xT = xT_ref[...]
        n = jnp.sqrt(jnp.sum(xT * xT, axis=0, keepdims=True))
        xnT = xT / (n + 1e-12)
        xeT_ref[...] = xnT * _C2
        down_ref[...] = jnp.zeros_like(down_ref)
        lc = lc_ref[...]
        lr = lr_ref[0:1, :]
        slot = lr - ((lr >> 3) << 3)
        gT = jnp.zeros((_D, _B), jnp.float32)
        for r in range(8):
            gT = gT + jnp.where(slot == r, g8r_ref[r], 0.0)
        gT_ref[...] = gT
        u = _MOM * gT + (1.0 - _MOM) * xnT
        un = jnp.sqrt(jnp.sum(u * u, axis=0, keepdims=True))
        u = u / (un + 1e-12)
        gb = (lr >> 3) << 3
        ii = lax.broadcasted_iota(jnp.int32, (_B, _B), 0)
        for r in range(8):
            eq_r = lc == (gb + r)
            lo_r = jnp.max(jnp.where(eq_r, ii, -1), axis=0, keepdims=True)
            q_r = (ii == lo_r).astype(jnp.float32)
            val_r = jnp.dot(u, q_r)
            t8_ref[r * _D:(r + 1) * _D, :] = jnp.where(
                lo_r >= 0, val_r, g8r_ref[r])

    sm = jnp.where(i > 0, s_ref[...], -1e30)
    e = jnp.exp2(sm)
    down_ref[...] += jnp.sum(e.reshape(_BR // 8, 8, _B), axis=0)
    fblk = f_ref[...]
    mem_ref[...] = fblk
    s_ref[...] = jnp.dot(fblk, xeT_ref[...])

    @pl.when(i == _NB)
    def _fin():
        dots = jnp.sum(xeT_ref[...] * gT_ref[...], axis=0, keepdims=True)
        down = jnp.sum(down_ref[...], axis=0, keepdims=True)
        p_ref[...] = jnp.exp2(dots) / down


_main_call = pl.pallas_call(
    _main_body,
    grid=(_NB + 1,),
    in_specs=[
        pl.BlockSpec((_D, _B), lambda i: (0, 0)),
        pl.BlockSpec((8, _D, _B), lambda i: (0, 0, 0)),
        pl.BlockSpec((_B, 1), lambda i: (0, 0)),
        pl.BlockSpec((8, _B), lambda i: (0, 0)),
        pl.BlockSpec((_BR, _D), lambda i: (jnp.minimum(i, _NB - 1), 0)),
    ],
    out_specs=[
        pl.BlockSpec((1, _B), lambda i: (0, 0)),
        pl.BlockSpec((8 * _D, _B), lambda i: (0, 0)),
        pl.BlockSpec((_BR, _D), lambda i: (jnp.minimum(i, _NB - 1), 0)),
    ],
    out_shape=[
        jax.ShapeDtypeStruct((1, _B), jnp.float32),
        jax.ShapeDtypeStruct((8 * _D, _B), jnp.float32),
        jax.ShapeDtypeStruct((_V, _D), jnp.float32),
    ],
    scratch_shapes=[
        pltpu.VMEM((_D, _B), jnp.float32),
        pltpu.VMEM((_D, _B), jnp.float32),
        pltpu.VMEM((_BR, _B), jnp.float32),
        pltpu.VMEM((8, _B), jnp.float32),
    ],
    compiler_params=pltpu.CompilerParams(
        dimension_semantics=("arbitrary",),
    ),
)


_S_NW = 32
_S_PER = _B // _S_NW


def _scatter_body(upd_ref, lab_ref, mem_ref, idx_v, rows_v, sem):
    c = lax.axis_index("c")
    s = lax.axis_index("s")
    wid = s * 2 + c
    i0 = wid * _S_PER
    pltpu.sync_copy(lab_ref.at[pl.ds(i0, _S_PER)], idx_v)
    pltpu.sync_copy(upd_ref.at[pl.ds(i0 * 8, _S_PER * 8)], rows_v)

    for grp in range(_S_PER // 16):
        lab_vec = idx_v[pl.ds(grp * 16, 16)]
        for j in range(16):
            k = grp * 16 + j
            tb = pl.multiple_of((lab_vec[j] >> 3) * 8, 8)
            pltpu.async_copy(rows_v.at[pl.ds(k * 8, 8)],
                             mem_ref.at[pl.ds(tb, 8)], sem)
    pltpu.make_async_copy(upd_ref.at[pl.ds(0, _S_PER * 8)], rows_v,
                          sem).wait()


@functools.cache
def _sc_scatter():
    return pl.kernel(
        _scatter_body,
        out_type=(),
        mesh=plsc.VectorSubcoreMesh(
            core_axis_name="c", subcore_axis_name="s", num_cores=2
        ),
        scratch_types=[
            pltpu.VMEM((_S_PER,), jnp.int32),
            pltpu.VMEM((_S_PER * 8, _D), jnp.float32),
            pltpu.SemaphoreType.DMA,
        ],
    )


def kernel(inputs, labels, features):
    labels = labels.astype(jnp.int32)
    g8 = _sc_gather()(labels, features)
    g8r = jnp.transpose(g8.reshape(_B, 8, _D), (1, 2, 0))
    p2, t8, mem_copy = _main_call(
        inputs.T,
        g8r,
        labels[:, None],
        jnp.broadcast_to(labels[None, :], (8, _B)),
        features,
    )
    upd8 = t8.T.reshape(_B * 8, _D)
    mem_ref = jax.new_ref(mem_copy)
    _sc_scatter()(upd8, labels, mem_ref)
    new_mem = jax.freeze(mem_ref)
    return p2.reshape(_B), new_mem


# speedup vs baseline: 2.4653x; 1.0693x over previous
import functools

import jax
import jax.numpy as jnp
from jax import lax
from jax.experimental import pallas as pl
from jax.experimental.pallas import tpu as pltpu
from jax.experimental.pallas import tpu_sc as plsc

_TEMP = 0.1
_MOM = 0.1
_V = 100000
_D = 64
_B = 1024
_BR = 4000
_NB = _V // _BR
_C2 = float(1.4426950408889634 / _TEMP)


_G_NW = 32
_G_PER = _B // _G_NW


def _gather_body(lab_ref, feat_ref, g8_ref, idx_v, rows_v, sem):
    c = lax.axis_index("c")
    s = lax.axis_index("s")
    wid = s * 2 + c
    base = wid * _G_PER
    pltpu.sync_copy(lab_ref.at[pl.ds(base, _G_PER)], idx_v)

    for grp in range(_G_PER // 16):
        lab_vec = idx_v[pl.ds(grp * 16, 16)]
        for j in range(16):
            k = grp * 16 + j
            tb = pl.multiple_of((lab_vec[j] >> 3) * 8, 8)
            pltpu.async_copy(feat_ref.at[pl.ds(tb, 8)],
                             rows_v.at[pl.ds(k * 8, 8)], sem)
    pltpu.make_async_copy(feat_ref.at[pl.ds(0, _G_PER * 8)], rows_v,
                          sem).wait()
    pltpu.sync_copy(rows_v, g8_ref.at[pl.ds(base * 8, _G_PER * 8)])


@functools.cache
def _sc_gather():
    return pl.kernel(
        _gather_body,
        out_type=jax.ShapeDtypeStruct((_B * 8, _D), jnp.float32),
        mesh=plsc.VectorSubcoreMesh(core_axis_name="c", subcore_axis_name="s"),
        scratch_types=[
            pltpu.VMEM((_G_PER,), jnp.int32),
            pltpu.VMEM((_G_PER * 8, _D), jnp.float32),
            pltpu.SemaphoreType.DMA,
        ],
    )


def _main_body(xT_ref, g8r_ref, lc_ref, lr_ref, f_ref, p_ref, t8_ref,
               mem_ref, xeT_ref, gT_ref, s_ref, down_ref):
    i = pl.program_id(0)

    @pl.when(i == 0)
    def _init():
        xT = xT_ref[...]
        n = jnp.sqrt(jnp.sum(xT * xT, axis=0, keepdims=True))
        xnT = xT / (n + 1e-12)
        xeT_ref[...] = xnT * _C2
        down_ref[...] = jnp.zeros_like(down_ref)
        lc = lc_ref[...]
        lr = lr_ref[0:1, :]
        slot = lr - ((lr >> 3) << 3)
        gT = jnp.zeros((_D, _B), jnp.float32)
        for r in range(8):
            gT = gT + jnp.where(slot == r, g8r_ref[r], 0.0)
        gT_ref[...] = gT
        u = _MOM * gT + (1.0 - _MOM) * xnT
        un = jnp.sqrt(jnp.sum(u * u, axis=0, keepdims=True))
        u = u / (un + 1e-12)
        gb = (lr >> 3) << 3
        ii = lax.broadcasted_iota(jnp.int32, (_B, _B), 0)
        for r in range(8):
            eq_r = lc == (gb + r)
            lo_r = jnp.max(jnp.where(eq_r, ii, -1), axis=0, keepdims=True)
            q_r = (ii == lo_r).astype(jnp.float32)
            val_r = jnp.dot(u, q_r)
            t8_ref[r * _D:(r + 1) * _D, :] = jnp.where(
                lo_r >= 0, val_r, g8r_ref[r])

    sm = jnp.where(i > 0, s_ref[...], -1e30)
    e = jnp.exp2(sm)
    down_ref[...] += jnp.sum(e.reshape(_BR // 8, 8, _B), axis=0)
    fblk = f_ref[...]
    mem_ref[...] = fblk
    s_ref[...] = jnp.dot(fblk, xeT_ref[...])

    @pl.when(i == _NB)
    def _fin():
        dots = jnp.sum(xeT_ref[...] * gT_ref[...], axis=0, keepdims=True)
        down = jnp.sum(down_ref[...], axis=0, keepdims=True)
        p_ref[...] = jnp.exp2(dots) / down


_main_call = pl.pallas_call(
    _main_body,
    grid=(_NB + 1,),
    in_specs=[
        pl.BlockSpec((_D, _B), lambda i: (0, 0)),
        pl.BlockSpec((8, _D, _B), lambda i: (0, 0, 0)),
        pl.BlockSpec((_B, 1), lambda i: (0, 0)),
        pl.BlockSpec((8, _B), lambda i: (0, 0)),
        pl.BlockSpec((_BR, _D), lambda i: (jnp.minimum(i, _NB - 1), 0)),
    ],
    out_specs=[
        pl.BlockSpec((1, _B), lambda i: (0, 0)),
        pl.BlockSpec((8 * _D, _B), lambda i: (0, 0)),
        pl.BlockSpec((_BR, _D), lambda i: (jnp.minimum(i, _NB - 1), 0)),
    ],
    out_shape=[
        jax.ShapeDtypeStruct((1, _B), jnp.float32),
        jax.ShapeDtypeStruct((8 * _D, _B), jnp.float32),
        jax.ShapeDtypeStruct((_V, _D), jnp.float32),
    ],
    scratch_shapes=[
        pltpu.VMEM((_D, _B), jnp.float32),
        pltpu.VMEM((_D, _B), jnp.float32),
        pltpu.VMEM((_BR, _B), jnp.float32),
        pltpu.VMEM((8, _B), jnp.float32),
    ],
    compiler_params=pltpu.CompilerParams(
        dimension_semantics=("arbitrary",),
    ),
)


_S_NW = 32
_S_PER = _B // _S_NW


def _scatter_body(upd_ref, lab_ref, mem_ref, idx_v, rows_v, sem):
    c = lax.axis_index("c")
    s = lax.axis_index("s")
    wid = s * 2 + c
    i0 = wid * _S_PER
    pltpu.sync_copy(lab_ref.at[pl.ds(i0, _S_PER)], idx_v)
    pltpu.sync_copy(upd_ref.at[pl.ds(i0 * 8, _S_PER * 8)], rows_v)

    for grp in range(_S_PER // 16):
        lab_vec = idx_v[pl.ds(grp * 16, 16)]
        for j in range(16):
            k = grp * 16 + j
            tb = pl.multiple_of((lab_vec[j] >> 3) * 8, 8)
            pltpu.async_copy(rows_v.at[pl.ds(k * 8, 8)],
                             mem_ref.at[pl.ds(tb, 8)], sem)
    pltpu.make_async_copy(upd_ref.at[pl.ds(0, _S_PER * 8)], rows_v,
                          sem).wait()


@functools.cache
def _sc_scatter():
    return pl.kernel(
        _scatter_body,
        out_type=(),
        mesh=plsc.VectorSubcoreMesh(
            core_axis_name="c", subcore_axis_name="s", num_cores=2
        ),
        scratch_types=[
            pltpu.VMEM((_S_PER,), jnp.int32),
            pltpu.VMEM((_S_PER * 8, _D), jnp.float32),
            pltpu.SemaphoreType.DMA,
        ],
    )


def kernel(inputs, labels, features):
    labels = labels.astype(jnp.int32)
    g8 = _sc_gather()(labels, features)
    g8r = jnp.transpose(g8.reshape(_B, 8, _D), (1, 2, 0))
    p2, t8, mem_copy = _main_call(
        inputs.T,
        g8r,
        labels[:, None],
        jnp.broadcast_to(labels[None, :], (8, _B)),
        features,
    )
    upd8 = t8.T.reshape(_B * 8, _D)
    mem_ref = jax.new_ref(mem_copy)
    _sc_scatter()(upd8, labels, mem_ref)
    new_mem = jax.freeze(mem_ref)
    return p2.reshape(_B), new_mem


# speedup vs baseline: 2.4867x; 1.0086x over previous
import functools

import jax
import jax.numpy as jnp
from jax import lax
from jax.experimental import pallas as pl
from jax.experimental.pallas import tpu as pltpu
from jax.experimental.pallas import tpu_sc as plsc

_TEMP = 0.1
_MOM = 0.1
_V = 100000
_D = 64
_B = 1024
_BR = 5000
_NB = _V // _BR
_C2 = float(1.4426950408889634 / _TEMP)


_G_NW = 32
_G_PER = _B // _G_NW


def _gather_body(lab_ref, feat_ref, g8_ref, idx_v, rows_v, sem):
    c = lax.axis_index("c")
    s = lax.axis_index("s")
    wid = s * 2 + c
    base = wid * _G_PER
    pltpu.sync_copy(lab_ref.at[pl.ds(base, _G_PER)], idx_v)

    for grp in range(_G_PER // 16):
        lab_vec = idx_v[pl.ds(grp * 16, 16)]
        for j in range(16):
            k = grp * 16 + j
            tb = pl.multiple_of((lab_vec[j] >> 3) * 8, 8)
            pltpu.async_copy(feat_ref.at[pl.ds(tb, 8)],
                             rows_v.at[pl.ds(k * 8, 8)], sem)
    pltpu.make_async_copy(feat_ref.at[pl.ds(0, _G_PER * 8)], rows_v,
                          sem).wait()
    pltpu.sync_copy(rows_v, g8_ref.at[pl.ds(base * 8, _G_PER * 8)])


@functools.cache
def _sc_gather():
    return pl.kernel(
        _gather_body,
        out_type=jax.ShapeDtypeStruct((_B * 8, _D), jnp.float32),
        mesh=plsc.VectorSubcoreMesh(core_axis_name="c", subcore_axis_name="s"),
        scratch_types=[
            pltpu.VMEM((_G_PER,), jnp.int32),
            pltpu.VMEM((_G_PER * 8, _D), jnp.float32),
            pltpu.SemaphoreType.DMA,
        ],
    )


def _main_body(xT_ref, g8r_ref, lc_ref, lr_ref, f_ref, p_ref, t8_ref,
               mem_ref, xeT_ref, gT_ref, s_ref, down_ref):
    i = pl.program_id(0)

    @pl.when(i == 0)
    def _init():
        xT = xT_ref[...]
        n = jnp.sqrt(jnp.sum(xT * xT, axis=0, keepdims=True))
        xnT = xT / (n + 1e-12)
        xeT_ref[...] = xnT * _C2
        down_ref[...] = jnp.zeros_like(down_ref)
        lc = lc_ref[...]
        lr = lr_ref[0:1, :]
        slot = lr - ((lr >> 3) << 3)
        gT = jnp.zeros((_D, _B), jnp.float32)
        for r in range(8):
            gT = gT + jnp.where(slot == r, g8r_ref[r], 0.0)
        gT_ref[...] = gT
        u = _MOM * gT + (1.0 - _MOM) * xnT
        un = jnp.sqrt(jnp.sum(u * u, axis=0, keepdims=True))
        u = u / (un + 1e-12)
        gb = (lr >> 3) << 3
        ii = lax.broadcasted_iota(jnp.int32, (_B, _B), 0)
        for r in range(8):
            eq_r = lc == (gb + r)
            lo_r = jnp.max(jnp.where(eq_r, ii, -1), axis=0, keepdims=True)
            q_r = (ii == lo_r).astype(jnp.float32)
            val_r = jnp.dot(u, q_r)
            t8_ref[r * _D:(r + 1) * _D, :] = jnp.where(
                lo_r >= 0, val_r, g8r_ref[r])

    sm = jnp.where(i > 0, s_ref[...], -1e30)
    e = jnp.exp2(sm)
    down_ref[...] += jnp.sum(e.reshape(_BR // 8, 8, _B), axis=0)
    fblk = f_ref[...]
    mem_ref[...] = fblk
    s_ref[...] = jnp.dot(fblk, xeT_ref[...])

    @pl.when(i == _NB)
    def _fin():
        dots = jnp.sum(xeT_ref[...] * gT_ref[...], axis=0, keepdims=True)
        down = jnp.sum(down_ref[...], axis=0, keepdims=True)
        p_ref[...] = jnp.exp2(dots) / down


_main_call = pl.pallas_call(
    _main_body,
    grid=(_NB + 1,),
    in_specs=[
        pl.BlockSpec((_D, _B), lambda i: (0, 0)),
        pl.BlockSpec((8, _D, _B), lambda i: (0, 0, 0)),
        pl.BlockSpec((_B, 1), lambda i: (0, 0)),
        pl.BlockSpec((8, _B), lambda i: (0, 0)),
        pl.BlockSpec((_BR, _D), lambda i: (jnp.minimum(i, _NB - 1), 0)),
    ],
    out_specs=[
        pl.BlockSpec((1, _B), lambda i: (0, 0)),
        pl.BlockSpec((8 * _D, _B), lambda i: (0, 0)),
        pl.BlockSpec((_BR, _D), lambda i: (jnp.minimum(i, _NB - 1), 0)),
    ],
    out_shape=[
        jax.ShapeDtypeStruct((1, _B), jnp.float32),
        jax.ShapeDtypeStruct((8 * _D, _B), jnp.float32),
        jax.ShapeDtypeStruct((_V, _D), jnp.float32),
    ],
    scratch_shapes=[
        pltpu.VMEM((_D, _B), jnp.float32),
        pltpu.VMEM((_D, _B), jnp.float32),
        pltpu.VMEM((_BR, _B), jnp.float32),
        pltpu.VMEM((8, _B), jnp.float32),
    ],
    compiler_params=pltpu.CompilerParams(
        dimension_semantics=("arbitrary",),
    ),
)


_S_NW = 32
_S_PER = _B // _S_NW


def _scatter_body(upd_ref, lab_ref, mem_ref, idx_v, rows_v, sem):
    c = lax.axis_index("c")
    s = lax.axis_index("s")
    wid = s * 2 + c
    i0 = wid * _S_PER
    pltpu.sync_copy(lab_ref.at[pl.ds(i0, _S_PER)], idx_v)
    pltpu.sync_copy(upd_ref.at[pl.ds(i0 * 8, _S_PER * 8)], rows_v)

    for grp in range(_S_PER // 16):
        lab_vec = idx_v[pl.ds(grp * 16, 16)]
        for j in range(16):
            k = grp * 16 + j
            tb = pl.multiple_of((lab_vec[j] >> 3) * 8, 8)
            pltpu.async_copy(rows_v.at[pl.ds(k * 8, 8)],
                             mem_ref.at[pl.ds(tb, 8)], sem)
    pltpu.make_async_copy(upd_ref.at[pl.ds(0, _S_PER * 8)], rows_v,
                          sem).wait()


@functools.cache
def _sc_scatter():
    return pl.kernel(
        _scatter_body,
        out_type=(),
        mesh=plsc.VectorSubcoreMesh(
            core_axis_name="c", subcore_axis_name="s", num_cores=2
        ),
        scratch_types=[
            pltpu.VMEM((_S_PER,), jnp.int32),
            pltpu.VMEM((_S_PER * 8, _D), jnp.float32),
            pltpu.SemaphoreType.DMA,
        ],
    )


def kernel(inputs, labels, features):
    labels = labels.astype(jnp.int32)
    g8 = _sc_gather()(labels, features)
    g8r = jnp.transpose(g8.reshape(_B, 8, _D), (1, 2, 0))
    p2, t8, mem_copy = _main_call(
        inputs.T,
        g8r,
        labels[:, None],
        jnp.broadcast_to(labels[None, :], (8, _B)),
        features,
    )
    upd8 = t8.T.reshape(_B * 8, _D)
    mem_ref = jax.new_ref(mem_copy)
    _sc_scatter()(upd8, labels, mem_ref)
    new_mem = jax.freeze(mem_ref)
    return p2.reshape(_B), new_mem
